# Initial kernel scaffold; baseline (speedup 1.0000x reference)
import jax
import jax.numpy as jnp
from jax.experimental import pallas as pl


def kernel(clause_var_idx, clause_sign, clause_ids, params):
    raise NotImplementedError("write your pallas kernel here")


# speedup vs baseline: 4.6226x; 4.6226x over previous
import functools
import jax
import jax.numpy as jnp
from jax import lax
from jax.experimental import pallas as pl

NV = 50000
NC = 200000
NL = 600000
F = 128
ROUNDS = 4

BV = 1000
BC = 1000


def _leaky(x):
    return jnp.where(x >= 0, x, 0.2 * x)


def _softplus(x):
    return jnp.logaddexp(x, 0.0)


def _query_body(v_ref, w1, b1, w2, b2, w3, b3, out_ref):
    x = v_ref[...]
    h = _leaky(jnp.dot(x, w1[...]) + b1[...])
    h = _leaky(jnp.dot(h, w2[...]) + b2[...])
    q = jnp.dot(h, w3[...]) + b3[...]
    out_ref[0] = _softplus(q)
    out_ref[1] = _softplus(-q)


def _query_table(variables, qw):
    (w1, b1), (w2, b2), (w3, b3) = qw
    wspecs = [pl.BlockSpec(w.shape, lambda i: (0,) * w.ndim) for w in
              (w1, b1.reshape(1, F), w2, b2.reshape(1, F), w3, b3.reshape(1, F))]
    out = pl.pallas_call(
        _query_body,
        grid=(NV // BV,),
        in_specs=[pl.BlockSpec((BV, F), lambda i: (i, 0))] + wspecs,
        out_specs=pl.BlockSpec((2, BV, F), lambda i: (0, i, 0)),
        out_shape=jax.ShapeDtypeStruct((2, NV, F), jnp.float32),
    )(variables, w1, b1.reshape(1, F), w2, b2.reshape(1, F), w3, b3.reshape(1, F))
    return out.reshape(2 * NV, F)


def _clause_body(cv_ref, pw1, pb1, pw2, pb2, pw3, pb3,
                 nw1, nb1, nw2, nb2, nw3, nb3, out_ref):
    cl = jnp.exp(-cv_ref[...])
    hp = _leaky(jnp.dot(cl, pw1[...]) + pb1[...])
    hp = _leaky(jnp.dot(hp, pw2[...]) + pb2[...])
    lp = jnp.dot(hp, pw3[...]) + pb3[...]
    hn = _leaky(jnp.dot(cl, nw1[...]) + nb1[...])
    hn = _leaky(jnp.dot(hn, nw2[...]) + nb2[...])
    ln = jnp.dot(hn, nw3[...]) + nb3[...]
    z = jnp.zeros_like(lp)
    out_ref[0] = jnp.concatenate([lp, z], axis=-1)
    out_ref[1] = jnp.concatenate([z, ln], axis=-1)


def _clause_table(cval, pw, nw):
    flat = []
    for (w, b) in pw + nw:
        flat.append(w)
        flat.append(b.reshape(1, -1))
    wspecs = [pl.BlockSpec(a.shape, lambda i: (0,) * a.ndim) for a in flat]
    out = pl.pallas_call(
        _clause_body,
        grid=(NC // BC,),
        in_specs=[pl.BlockSpec((BC, F), lambda i: (i, 0))] + wspecs,
        out_specs=pl.BlockSpec((2, BC, F), lambda i: (0, i, 0)),
        out_shape=jax.ShapeDtypeStruct((2, NC, F), jnp.float32),
    )(cval, *flat)
    return out.reshape(2 * NC, F)


def _update_body(v_ref, vu_ref, fw1, fb1, fw2, fb2, fw3, fb3,
                 uw1, ub1, uw2, ub2, uw3, ub3, gam, bet,
                 ow1, ob1, ow2, ob2, ow3, ob3, vout_ref, log_ref):
    v = v_ref[...]
    unit = jnp.concatenate([v, vu_ref[...]], axis=-1)
    h = _leaky(jnp.dot(unit, fw1[...]) + fb1[...])
    h = _leaky(jnp.dot(h, fw2[...]) + fb2[...])
    f = jax.nn.sigmoid(jnp.dot(h, fw3[...]) + fb3[...])
    h = _leaky(jnp.dot(unit, uw1[...]) + ub1[...])
    h = _leaky(jnp.dot(h, uw2[...]) + ub2[...])
    n = jnp.dot(h, uw3[...]) + ub3[...]
    v2 = (1.0 - f) * v + f * n
    mu = jnp.mean(v2, axis=-1, keepdims=True)
    var = jnp.mean((v2 - mu) ** 2, axis=-1, keepdims=True)
    v2 = (v2 - mu) / jnp.sqrt(var + 1e-3) * gam[...] + bet[...]
    vout_ref[...] = v2
    h = _leaky(jnp.dot(v2, ow1[...]) + ob1[...])
    h = _leaky(jnp.dot(h, ow2[...]) + ob2[...])
    log_ref[...] = jnp.dot(h, ow3[...]) + ob3[...]


def _update(variables, vu, fw, uw, gamma, beta, ow):
    (ow3, ob3) = ow[-1]
    ow3p = jnp.zeros((F, F), jnp.float32).at[:, 0:1].set(ow3)
    ob3p = jnp.zeros((1, F), jnp.float32).at[0, 0].set(ob3[0])
    flat = []
    for (w, b) in fw + uw:
        flat.append(w)
        flat.append(b.reshape(1, -1))
    flat.append(gamma.reshape(1, F))
    flat.append(beta.reshape(1, F))
    for (w, b) in ow[:-1]:
        flat.append(w)
        flat.append(b.reshape(1, -1))
    flat.append(ow3p)
    flat.append(ob3p)
    wspecs = [pl.BlockSpec(a.shape, lambda i: (0,) * a.ndim) for a in flat]
    vout, logf = pl.pallas_call(
        _update_body,
        grid=(NV // BV,),
        in_specs=[pl.BlockSpec((BV, F), lambda i: (i, 0)),
                  pl.BlockSpec((BV, F), lambda i: (i, 0))] + wspecs,
        out_specs=[pl.BlockSpec((BV, F), lambda i: (i, 0)),
                   pl.BlockSpec((BV, F), lambda i: (i, 0))],
        out_shape=[jax.ShapeDtypeStruct((NV, F), jnp.float32),
                   jax.ShapeDtypeStruct((NV, F), jnp.float32)],
    )(variables, vu, *flat)
    return vout, logf[:, 0:1]


@jax.jit
def _run(clause_var_idx, clause_sign, clause_ids, params):
    variables = 0.25 * jax.random.truncated_normal(
        jax.random.key(1), -2.0, 2.0, (NV, F), dtype=jnp.float32)

    idx_a = clause_var_idx + NV * (1 - clause_sign)
    idx_b = clause_ids + NC * (1 - clause_sign)

    step_logits = []
    v = variables
    for _ in range(ROUNDS):
        tq = _query_table(v, params['variables_query'])
        cval = jax.ops.segment_sum(tq[idx_a], clause_ids, num_segments=NC)
        tcl = _clause_table(cval, params['query_pos_inter'], params['query_neg_inter'])
        vu = jnp.zeros((NV, F), jnp.float32).at[clause_var_idx].add(tcl[idx_b])
        v, logit = _update(v, vu, params['forget_gate'], params['update_gate'],
                           params['ln_gamma'], params['ln_beta'],
                           params['variables_output'])
        step_logits.append(logit)
    return jnp.stack(step_logits, axis=0)


def kernel(clause_var_idx, clause_sign, clause_ids, params):
    return _run(clause_var_idx, clause_sign, clause_ids, params)
